# Initial kernel scaffold; baseline (speedup 1.0000x reference)
import jax
import jax.numpy as jnp
from jax.experimental import pallas as pl


def kernel(sess_emb, pool_emb):
    raise NotImplementedError("write your pallas kernel here")


# speedup vs baseline: 2.4220x; 2.4220x over previous
import functools

import jax
import jax.numpy as jnp
from jax import lax
from jax.experimental import pallas as pl
from jax.experimental.pallas import tpu as pltpu
from jax.experimental.pallas import tpu_sc as plsc

H = 128
K = 3
QB = 1024
PB = 512
NEG = -3e38


def _tc_body(npool, sess_ref, pool_ref, cosk_ref, idx_ref,
             v1, v2, v3, i1, i2, i3, tsum):
    pi = pl.program_id(1)
    np_total = pl.num_programs(1)

    @pl.when(pi == 0)
    def _init():
        neg = jnp.full(v1.shape, NEG, jnp.float32)
        v1[...] = neg
        v2[...] = neg
        v3[...] = neg
        zi = jnp.zeros(i1.shape, jnp.int32)
        i1[...] = zi
        i2[...] = zi
        i3[...] = zi
        tsum[...] = jnp.zeros(tsum.shape, jnp.float32)

    q = sess_ref[...]
    k = pool_ref[...]
    fz = lax.dot_general(k.astype(jnp.bfloat16), q.astype(jnp.bfloat16),
                         (((1,), (1,)), ((), ())),
                         preferred_element_type=jnp.float32)
    rfr = 1.0 / jnp.sqrt(jnp.sum(k * k + 1e-6, axis=1, keepdims=True))
    qsq = (q * q + 1e-6).astype(jnp.bfloat16)
    one = jnp.ones((1, H), jnp.bfloat16)
    rfl = 1.0 / jnp.sqrt(lax.dot_general(
        one, qsq, (((1,), (1,)), ((), ())),
        preferred_element_type=jnp.float32))

    s = fz * rfr
    row = lax.broadcasted_iota(jnp.int32, s.shape, 0)
    s = jnp.where(row + pi * PB < npool, s, NEG)

    tsum[...] += jnp.sum(jnp.exp(s * rfl), axis=0, keepdims=True)

    work = s
    for t in range(K):
        m = jnp.max(work, axis=0, keepdims=True)
        a = jnp.min(jnp.where(work == m, row, PB), axis=0, keepdims=True)
        g = a + pi * PB
        if t < K - 1:
            work = jnp.where(row == a, NEG, work)
        gt1 = m > v1[...]
        gt2 = m > v2[...]
        gt3 = m > v3[...]
        v3[...] = jnp.where(gt2, v2[...], jnp.where(gt3, m, v3[...]))
        i3[...] = jnp.where(gt2, i2[...], jnp.where(gt3, g, i3[...]))
        v2[...] = jnp.where(gt1, v1[...], jnp.where(gt2, m, v2[...]))
        i2[...] = jnp.where(gt1, i1[...], jnp.where(gt2, g, i2[...]))
        v1[...] = jnp.where(gt1, m, v1[...])
        i1[...] = jnp.where(gt1, g, i1[...])

    @pl.when(pi == np_total - 1)
    def _fin():
        t = tsum[...]
        p1 = jnp.exp(v1[...] * rfl) / t
        p2 = jnp.exp(v2[...] * rfl) / t
        p3 = jnp.exp(v3[...] * rfl) / t
        mx = jnp.maximum(p1, jnp.maximum(p2, p3))
        e1 = jnp.exp(p1 - mx)
        e2 = jnp.exp(p2 - mx)
        e3 = jnp.exp(p3 - mx)
        z = e1 + e2 + e3
        cosk_ref[...] = jnp.concatenate([e1 / z, e2 / z, e3 / z], axis=0)
        idx_ref[...] = jnp.concatenate([i1[...], i2[...], i3[...]], axis=0)


def _topk_call(sess_emb, pool_pad, npool):
    nq, h = sess_emb.shape
    grid = (nq // QB, pool_pad.shape[0] // PB)
    return pl.pallas_call(
        functools.partial(_tc_body, npool),
        grid=grid,
        in_specs=[
            pl.BlockSpec((QB, h), lambda qi, pi: (qi, 0)),
            pl.BlockSpec((PB, h), lambda qi, pi: (pi, 0)),
        ],
        out_specs=[
            pl.BlockSpec((K, QB), lambda qi, pi: (0, qi)),
            pl.BlockSpec((K, QB), lambda qi, pi: (0, qi)),
        ],
        out_shape=[
            jax.ShapeDtypeStruct((K, nq), jnp.float32),
            jax.ShapeDtypeStruct((K, nq), jnp.int32),
        ],
        scratch_shapes=[
            pltpu.VMEM((1, QB), jnp.float32),
            pltpu.VMEM((1, QB), jnp.float32),
            pltpu.VMEM((1, QB), jnp.float32),
            pltpu.VMEM((1, QB), jnp.int32),
            pltpu.VMEM((1, QB), jnp.int32),
            pltpu.VMEM((1, QB), jnp.int32),
            pltpu.VMEM((1, QB), jnp.float32),
        ],
        compiler_params=pltpu.CompilerParams(
            dimension_semantics=("arbitrary", "arbitrary")),
    )(sess_emb, pool_pad)


def _sc_body(g_per_w, nc, pool_hbm, idx_hbm, w_hbm, outg_hbm, outn_hbm,
             idx_v, rows_v, acc_v, w_v, sem):
    wid = lax.axis_index("s") * nc + lax.axis_index("c")
    pltpu.sync_copy(idx_hbm.at[wid], idx_v)
    pltpu.sync_copy(w_hbm.at[pl.ds(g_per_w * wid, g_per_w)], w_v)
    copies = [
        pltpu.async_copy(pool_hbm.at[idx_v.at[j]],
                         rows_v.at[pl.ds(j * 128, 128)], sem)
        for j in range(K)
    ]
    for c in copies:
        c.wait()
    pltpu.sync_copy(rows_v, outg_hbm.at[pl.ds(g_per_w * wid, g_per_w)])

    def row_body(r, carry):
        for grp in range(H // 16):
            sl = pl.ds(grp * 16, 16)
            acc = (w_v[K * r, sl] * rows_v[K * r, sl]
                   + w_v[K * r + 1, sl] * rows_v[K * r + 1, sl]
                   + w_v[K * r + 2, sl] * rows_v[K * r + 2, sl])
            acc_v[r, sl] = acc
        return carry

    lax.fori_loop(0, g_per_w // K, row_body, 0)
    pltpu.sync_copy(acc_v, outn_hbm.at[pl.ds((g_per_w // K) * wid,
                                             g_per_w // K)])


def _gather_call(pool_emb, topk_idx, cos_topk):
    nq = topk_idx.shape[0]
    h = pool_emb.shape[1]
    info = plsc.get_sparse_core_info()
    nc, ns = info.num_cores, info.num_subcores
    nw = nc * ns
    g_per_w = (nq * K) // nw
    idx3d = topk_idx.reshape(nw, K, 128)
    w_rep = jnp.broadcast_to(cos_topk.reshape(-1, 1), (nq * K, h))
    mesh = plsc.VectorSubcoreMesh(core_axis_name="c", subcore_axis_name="s")
    kfn = functools.partial(
        pl.kernel,
        mesh=mesh,
        out_type=[
            jax.ShapeDtypeStruct((nq * K, h), jnp.float32),
            jax.ShapeDtypeStruct((nq, h), jnp.float32),
        ],
        scratch_types=[
            pltpu.VMEM((K, 128), jnp.int32),
            pltpu.VMEM((g_per_w, h), jnp.float32),
            pltpu.VMEM((g_per_w // K, h), jnp.float32),
            pltpu.VMEM((g_per_w, h), jnp.float32),
            pltpu.SemaphoreType.DMA,
        ],
    )(functools.partial(_sc_body, g_per_w, nc))
    return kfn(pool_emb, idx3d, w_rep)


def kernel(sess_emb, pool_emb):
    npool = pool_emb.shape[0]
    npad = (-npool) % PB
    pool_pad = jnp.pad(pool_emb, ((0, npad), (0, 0)))
    cos_t, idx_t = _topk_call(sess_emb, pool_pad, npool)
    cos_topk, topk_idx = cos_t.T, idx_t.T
    gathered, neighbor = _gather_call(pool_emb, topk_idx, cos_topk)
    sess_topk = gathered.reshape(sess_emb.shape[0], K, H)
    return (neighbor, cos_topk, sess_topk)


# speedup vs baseline: 2.5148x; 1.0383x over previous
import functools

import jax
import jax.numpy as jnp
from jax import lax
from jax.experimental import pallas as pl
from jax.experimental.pallas import tpu as pltpu
from jax.experimental.pallas import tpu_sc as plsc

H = 128
K = 3
QB = 1024
PB = 1024
NEG = -3e38


def _tc_body(npool, sess_ref, pool_ref, cosk_ref, idx_ref,
             v1, v2, v3, i1, i2, i3, tsum):
    pi = pl.program_id(1)
    np_total = pl.num_programs(1)

    @pl.when(pi == 0)
    def _init():
        neg = jnp.full(v1.shape, NEG, jnp.float32)
        v1[...] = neg
        v2[...] = neg
        v3[...] = neg
        zi = jnp.zeros(i1.shape, jnp.int32)
        i1[...] = zi
        i2[...] = zi
        i3[...] = zi
        tsum[...] = jnp.zeros(tsum.shape, jnp.float32)

    q = sess_ref[...]
    k = pool_ref[...]
    fz = lax.dot_general(k.astype(jnp.bfloat16), q.astype(jnp.bfloat16),
                         (((1,), (1,)), ((), ())),
                         preferred_element_type=jnp.float32)
    rfr = 1.0 / jnp.sqrt(jnp.sum(k * k + 1e-6, axis=1, keepdims=True))
    qsq = (q * q + 1e-6).astype(jnp.bfloat16)
    one = jnp.ones((1, H), jnp.bfloat16)
    rfl = 1.0 / jnp.sqrt(lax.dot_general(
        one, qsq, (((1,), (1,)), ((), ())),
        preferred_element_type=jnp.float32))

    s = fz * rfr
    row = lax.broadcasted_iota(jnp.int32, s.shape, 0)
    s = jnp.where(row + pi * PB < npool, s, NEG)

    tsum[...] += jnp.sum(jnp.exp(s * rfl), axis=0, keepdims=True)

    work = s
    for t in range(K):
        m = jnp.max(work, axis=0, keepdims=True)
        a = jnp.min(jnp.where(work == m, row, PB), axis=0, keepdims=True)
        g = a + pi * PB
        if t < K - 1:
            work = jnp.where(row == a, NEG, work)
        gt1 = m > v1[...]
        gt2 = m > v2[...]
        gt3 = m > v3[...]
        v3[...] = jnp.where(gt2, v2[...], jnp.where(gt3, m, v3[...]))
        i3[...] = jnp.where(gt2, i2[...], jnp.where(gt3, g, i3[...]))
        v2[...] = jnp.where(gt1, v1[...], jnp.where(gt2, m, v2[...]))
        i2[...] = jnp.where(gt1, i1[...], jnp.where(gt2, g, i2[...]))
        v1[...] = jnp.where(gt1, m, v1[...])
        i1[...] = jnp.where(gt1, g, i1[...])

    @pl.when(pi == np_total - 1)
    def _fin():
        t = tsum[...]
        p1 = jnp.exp(v1[...] * rfl) / t
        p2 = jnp.exp(v2[...] * rfl) / t
        p3 = jnp.exp(v3[...] * rfl) / t
        mx = jnp.maximum(p1, jnp.maximum(p2, p3))
        e1 = jnp.exp(p1 - mx)
        e2 = jnp.exp(p2 - mx)
        e3 = jnp.exp(p3 - mx)
        z = e1 + e2 + e3
        cosk_ref[...] = jnp.concatenate([e1 / z, e2 / z, e3 / z], axis=0)
        idx_ref[...] = jnp.concatenate([i1[...], i2[...], i3[...]], axis=0)


def _topk_call(sess_emb, pool_pad, npool):
    nq, h = sess_emb.shape
    grid = (nq // QB, pool_pad.shape[0] // PB)
    return pl.pallas_call(
        functools.partial(_tc_body, npool),
        grid=grid,
        in_specs=[
            pl.BlockSpec((QB, h), lambda qi, pi: (qi, 0)),
            pl.BlockSpec((PB, h), lambda qi, pi: (pi, 0)),
        ],
        out_specs=[
            pl.BlockSpec((K, QB), lambda qi, pi: (0, qi)),
            pl.BlockSpec((K, QB), lambda qi, pi: (0, qi)),
        ],
        out_shape=[
            jax.ShapeDtypeStruct((K, nq), jnp.float32),
            jax.ShapeDtypeStruct((K, nq), jnp.int32),
        ],
        scratch_shapes=[
            pltpu.VMEM((1, QB), jnp.float32),
            pltpu.VMEM((1, QB), jnp.float32),
            pltpu.VMEM((1, QB), jnp.float32),
            pltpu.VMEM((1, QB), jnp.int32),
            pltpu.VMEM((1, QB), jnp.int32),
            pltpu.VMEM((1, QB), jnp.int32),
            pltpu.VMEM((1, QB), jnp.float32),
        ],
        compiler_params=pltpu.CompilerParams(
            dimension_semantics=("arbitrary", "arbitrary")),
    )(sess_emb, pool_pad)


def _sc_body(g_per_w, nc, pool_hbm, idx_hbm, w_hbm, outg_hbm, outn_hbm,
             idx_v, rows_v, acc_v, w_v, sem):
    wid = lax.axis_index("s") * nc + lax.axis_index("c")
    pltpu.sync_copy(idx_hbm.at[wid], idx_v)
    pltpu.sync_copy(w_hbm.at[pl.ds(g_per_w * wid, g_per_w)], w_v)
    copies = [
        pltpu.async_copy(pool_hbm.at[idx_v.at[j]],
                         rows_v.at[pl.ds(j * 128, 128)], sem)
        for j in range(K)
    ]
    for c in copies:
        c.wait()
    pltpu.sync_copy(rows_v, outg_hbm.at[pl.ds(g_per_w * wid, g_per_w)])

    def row_body(r, carry):
        for grp in range(H // 16):
            sl = pl.ds(grp * 16, 16)
            acc = (w_v[K * r, sl] * rows_v[K * r, sl]
                   + w_v[K * r + 1, sl] * rows_v[K * r + 1, sl]
                   + w_v[K * r + 2, sl] * rows_v[K * r + 2, sl])
            acc_v[r, sl] = acc
        return carry

    lax.fori_loop(0, g_per_w // K, row_body, 0)
    pltpu.sync_copy(acc_v, outn_hbm.at[pl.ds((g_per_w // K) * wid,
                                             g_per_w // K)])


def _gather_call(pool_emb, topk_idx, cos_topk):
    nq = topk_idx.shape[0]
    h = pool_emb.shape[1]
    info = plsc.get_sparse_core_info()
    nc, ns = info.num_cores, info.num_subcores
    nw = nc * ns
    g_per_w = (nq * K) // nw
    idx3d = topk_idx.reshape(nw, K, 128)
    w_rep = jnp.broadcast_to(cos_topk.reshape(-1, 1), (nq * K, h))
    mesh = plsc.VectorSubcoreMesh(core_axis_name="c", subcore_axis_name="s")
    kfn = functools.partial(
        pl.kernel,
        mesh=mesh,
        out_type=[
            jax.ShapeDtypeStruct((nq * K, h), jnp.float32),
            jax.ShapeDtypeStruct((nq, h), jnp.float32),
        ],
        scratch_types=[
            pltpu.VMEM((K, 128), jnp.int32),
            pltpu.VMEM((g_per_w, h), jnp.float32),
            pltpu.VMEM((g_per_w // K, h), jnp.float32),
            pltpu.VMEM((g_per_w, h), jnp.float32),
            pltpu.SemaphoreType.DMA,
        ],
    )(functools.partial(_sc_body, g_per_w, nc))
    return kfn(pool_emb, idx3d, w_rep)


def kernel(sess_emb, pool_emb):
    npool = pool_emb.shape[0]
    npad = (-npool) % PB
    pool_pad = jnp.pad(pool_emb, ((0, npad), (0, 0)))
    cos_t, idx_t = _topk_call(sess_emb, pool_pad, npool)
    cos_topk, topk_idx = cos_t.T, idx_t.T
    gathered, neighbor = _gather_call(pool_emb, topk_idx, cos_topk)
    sess_topk = gathered.reshape(sess_emb.shape[0], K, H)
    return (neighbor, cos_topk, sess_topk)


# speedup vs baseline: 3.0825x; 1.2257x over previous
import functools

import jax
import jax.numpy as jnp
from jax import lax
from jax.experimental import pallas as pl
from jax.experimental.pallas import tpu as pltpu
from jax.experimental.pallas import tpu_sc as plsc

H = 128
K = 3
QB = 1024
PB = 1000
NEG = -3e38


LOG2E = 1.4426950408889634


def _tc_body(sess_ref, pool_ref, cosk_ref, idx_ref,
             v1, v2, v3, i1, i2, i3, tsum):
    pi = pl.program_id(1)
    np_total = pl.num_programs(1)

    @pl.when(pi == 0)
    def _init():
        neg = jnp.full(v1.shape, NEG, jnp.float32)
        v1[...] = neg
        v2[...] = neg
        v3[...] = neg
        zi = jnp.zeros(i1.shape, jnp.float32)
        i1[...] = zi
        i2[...] = zi
        i3[...] = zi
        tsum[...] = jnp.zeros(tsum.shape, jnp.float32)

    q = sess_ref[...]
    k = pool_ref[...]
    fz = lax.dot_general(k.astype(jnp.bfloat16), q.astype(jnp.bfloat16),
                         (((1,), (1,)), ((), ())),
                         preferred_element_type=jnp.float32)
    rfr = 1.0 / jnp.sqrt(jnp.sum(k * k + 1e-6, axis=1, keepdims=True))
    qsq = (q * q + 1e-6).astype(jnp.bfloat16)
    one = jnp.ones((1, H), jnp.bfloat16)
    rfl = LOG2E / jnp.sqrt(lax.dot_general(
        one, qsq, (((1,), (1,)), ((), ())),
        preferred_element_type=jnp.float32))

    s = fz * rfr
    row = lax.broadcasted_iota(jnp.int32, s.shape, 0).astype(jnp.float32)

    tsum[...] += jnp.sum(jnp.exp2(s * rfl), axis=0, keepdims=True)

    work = s
    for t in range(K):
        m = jnp.max(work, axis=0, keepdims=True)
        a = jnp.min(jnp.where(work == m, row, float(PB)),
                    axis=0, keepdims=True)
        g = a + jnp.float32(pi * PB)
        if t < K - 1:
            work = jnp.where(row == a, NEG, work)
        gt1 = m > v1[...]
        gt2 = m > v2[...]
        gt3 = m > v3[...]
        v3[...] = jnp.where(gt2, v2[...], jnp.where(gt3, m, v3[...]))
        i3[...] = jnp.where(gt2, i2[...], jnp.where(gt3, g, i3[...]))
        v2[...] = jnp.where(gt1, v1[...], jnp.where(gt2, m, v2[...]))
        i2[...] = jnp.where(gt1, i1[...], jnp.where(gt2, g, i2[...]))
        v1[...] = jnp.where(gt1, m, v1[...])
        i1[...] = jnp.where(gt1, g, i1[...])

    @pl.when(pi == np_total - 1)
    def _fin():
        t = tsum[...]
        p1 = jnp.exp2(v1[...] * rfl) / t
        p2 = jnp.exp2(v2[...] * rfl) / t
        p3 = jnp.exp2(v3[...] * rfl) / t
        mx = jnp.maximum(p1, jnp.maximum(p2, p3))
        e1 = jnp.exp(p1 - mx)
        e2 = jnp.exp(p2 - mx)
        e3 = jnp.exp(p3 - mx)
        z = e1 + e2 + e3
        cosk_ref[...] = jnp.concatenate([e1 / z, e2 / z, e3 / z], axis=0)
        idx_ref[...] = jnp.concatenate(
            [i1[...], i2[...], i3[...]], axis=0).astype(jnp.int32)


def _topk_call(sess_emb, pool_emb):
    nq, h = sess_emb.shape
    grid = (nq // QB, pool_emb.shape[0] // PB)
    return pl.pallas_call(
        _tc_body,
        grid=grid,
        in_specs=[
            pl.BlockSpec((QB, h), lambda qi, pi: (qi, 0)),
            pl.BlockSpec((PB, h), lambda qi, pi: (pi, 0)),
        ],
        out_specs=[
            pl.BlockSpec((K, QB), lambda qi, pi: (0, qi)),
            pl.BlockSpec((K, QB), lambda qi, pi: (0, qi)),
        ],
        out_shape=[
            jax.ShapeDtypeStruct((K, nq), jnp.float32),
            jax.ShapeDtypeStruct((K, nq), jnp.int32),
        ],
        scratch_shapes=[
            pltpu.VMEM((1, QB), jnp.float32),
            pltpu.VMEM((1, QB), jnp.float32),
            pltpu.VMEM((1, QB), jnp.float32),
            pltpu.VMEM((1, QB), jnp.float32),
            pltpu.VMEM((1, QB), jnp.float32),
            pltpu.VMEM((1, QB), jnp.float32),
            pltpu.VMEM((1, QB), jnp.float32),
        ],
        compiler_params=pltpu.CompilerParams(
            dimension_semantics=("arbitrary", "arbitrary")),
    )(sess_emb, pool_emb)


def _sc_body(g_per_w, nc, pool_hbm, idx_hbm, w_hbm, outg_hbm, outn_hbm,
             idx_v, rows_v, acc_v, w_v, sem):
    wid = lax.axis_index("s") * nc + lax.axis_index("c")
    pltpu.sync_copy(idx_hbm.at[wid], idx_v)
    pltpu.sync_copy(w_hbm.at[pl.ds(g_per_w * wid, g_per_w)], w_v)
    copies = [
        pltpu.async_copy(pool_hbm.at[idx_v.at[j]],
                         rows_v.at[pl.ds(j * 128, 128)], sem)
        for j in range(K)
    ]
    for c in copies:
        c.wait()
    pltpu.sync_copy(rows_v, outg_hbm.at[pl.ds(g_per_w * wid, g_per_w)])

    def row_body(r, carry):
        for grp in range(H // 16):
            sl = pl.ds(grp * 16, 16)
            acc = (w_v[K * r, sl] * rows_v[K * r, sl]
                   + w_v[K * r + 1, sl] * rows_v[K * r + 1, sl]
                   + w_v[K * r + 2, sl] * rows_v[K * r + 2, sl])
            acc_v[r, sl] = acc
        return carry

    lax.fori_loop(0, g_per_w // K, row_body, 0)
    pltpu.sync_copy(acc_v, outn_hbm.at[pl.ds((g_per_w // K) * wid,
                                             g_per_w // K)])


def _gather_call(pool_emb, topk_idx, cos_topk):
    nq = topk_idx.shape[0]
    h = pool_emb.shape[1]
    info = plsc.get_sparse_core_info()
    nc, ns = info.num_cores, info.num_subcores
    nw = nc * ns
    g_per_w = (nq * K) // nw
    idx3d = topk_idx.reshape(nw, K, 128)
    w_rep = jnp.broadcast_to(cos_topk.reshape(-1, 1), (nq * K, h))
    mesh = plsc.VectorSubcoreMesh(core_axis_name="c", subcore_axis_name="s")
    kfn = functools.partial(
        pl.kernel,
        mesh=mesh,
        out_type=[
            jax.ShapeDtypeStruct((nq * K, h), jnp.float32),
            jax.ShapeDtypeStruct((nq, h), jnp.float32),
        ],
        scratch_types=[
            pltpu.VMEM((K, 128), jnp.int32),
            pltpu.VMEM((g_per_w, h), jnp.float32),
            pltpu.VMEM((g_per_w // K, h), jnp.float32),
            pltpu.VMEM((g_per_w, h), jnp.float32),
            pltpu.SemaphoreType.DMA,
        ],
    )(functools.partial(_sc_body, g_per_w, nc))
    return kfn(pool_emb, idx3d, w_rep)


def kernel(sess_emb, pool_emb):
    cos_t, idx_t = _topk_call(sess_emb, pool_emb)
    cos_topk, topk_idx = cos_t.T, idx_t.T
    gathered, neighbor = _gather_call(pool_emb, topk_idx, cos_topk)
    sess_topk = gathered.reshape(sess_emb.shape[0], K, H)
    return (neighbor, cos_topk, sess_topk)
